# baseline (speedup 1.0000x reference)
import functools

import jax
import jax.numpy as jnp
from jax import lax
from jax.experimental import pallas as pl
from jax.experimental.pallas import tpu as pltpu
from jax.experimental.pallas import tpu_sc as plsc

_USER_NUM = 1000000
_ITEM_NUM = 100000
_D = 32
_B = 16384

_NC = 2
_NS = 16
_NW = _NC * _NS
_BPW = _B // _NW
_L = 16
_NG = _BPW // _L
_CHUNK = 128
_NCHUNK = _BPW // _CHUNK
_COL = 128
_LASTCOL = (_USER_NUM // _COL) * _COL


def _sigmoid(x):
    return 1.0 / (1.0 + jnp.exp(-x))


def _item_body(item, diff_w, exerk_w, out_sd, out_disc,
               idx_c, diff_v, ek_v, sd_v, disc_v, sem, sem2):
    wid = lax.axis_index("s") * _NC + lax.axis_index("c")
    base = wid * _BPW

    for c in range(_NCHUNK):
        pltpu.sync_copy(item.at[pl.ds(base + c * _CHUNK, _CHUNK)], idx_c.at[c])
    for c in range(_NCHUNK):
        for k in range(_CHUNK // _L):
            sl = pl.ds(k * _L, _L)
            v = idx_c[c, sl]
            idx_c[c, sl] = jnp.minimum(jnp.maximum(v, 0), _ITEM_NUM - 1)

    copies = []
    for c in range(_NCHUNK):
        rs = pl.ds(c * _CHUNK, _CHUNK)
        copies.append(pltpu.async_copy(diff_w.at[idx_c.at[c]],
                                       diff_v.at[rs], sem))
        copies.append(pltpu.async_copy(exerk_w.at[idx_c.at[c]],
                                       ek_v.at[rs], sem2))
    for cp in copies:
        cp.wait()

    iota = lax.iota(jnp.int32, _L)
    zero = jnp.zeros((_L,), jnp.float32)

    def group(g, carry):
        sl = pl.ds(g * _L, _L)
        rid = g * _L + iota
        s0 = zero
        s1 = zero
        s2 = zero
        s3 = zero
        for d in range(0, _D, 4):
            s0 = s0 + _sigmoid(plsc.load_gather(
                diff_v, [rid, jnp.full((_L,), d, jnp.int32)]))
            s1 = s1 + _sigmoid(plsc.load_gather(
                diff_v, [rid, jnp.full((_L,), d + 1, jnp.int32)]))
            s2 = s2 + _sigmoid(plsc.load_gather(
                diff_v, [rid, jnp.full((_L,), d + 2, jnp.int32)]))
            s3 = s3 + _sigmoid(plsc.load_gather(
                diff_v, [rid, jnp.full((_L,), d + 3, jnp.int32)]))
        sd_v[sl] = (s0 + s1) + (s2 + s3)
        disc_v[sl] = 2.0 * _sigmoid(ek_v[sl])
        return carry

    lax.fori_loop(0, _NG, group, 0)

    pltpu.sync_copy(sd_v, out_sd.at[pl.ds(base, _BPW)])
    pltpu.sync_copy(disc_v, out_disc.at[pl.ds(base, _BPW)])


@functools.partial(
    pl.kernel,
    mesh=plsc.VectorSubcoreMesh(core_axis_name="c", subcore_axis_name="s"),
    compiler_params=pltpu.CompilerParams(
        needs_layout_passes=False, use_tc_tiling_on_sc=False),
    out_type=[
        jax.ShapeDtypeStruct((_B,), jnp.float32),
        jax.ShapeDtypeStruct((_B,), jnp.float32),
    ],
    scratch_types=[
        pltpu.VMEM((_NCHUNK, _CHUNK), jnp.int32),
        pltpu.VMEM((_BPW, _D), jnp.float32),
        pltpu.VMEM((_BPW,), jnp.float32),
        pltpu.VMEM((_BPW,), jnp.float32),
        pltpu.VMEM((_BPW,), jnp.float32),
        pltpu.SemaphoreType.DMA,
        pltpu.SemaphoreType.DMA,
    ],
)
def _item_sc(*refs):
    _item_body(*refs)


def _user_body(user, pro_t, stu_t, sd_in, disc_in, wvec,
               out_o, out_a0, out_a1, out_a2, out_a3,
               idx_u, pb0, pb1, pb2, pb3, pb4, pb5, pb6, pb7,
               pb8, pb9, pb10, pb11, pb12, pb13,
               sb0, sb1, sb2, sb3, sb4, sb5, sb6, sb7,
               sb8, sb9, sb10, sb11, sb12, sb13, stp, sts,
               sd_v, disc_v, o_v, a0_v, a1_v, a2_v, a3_v, wv,
               sem_p, sem_s):
    wid = lax.axis_index("s") * _NC + lax.axis_index("c")
    base = wid * _BPW

    pltpu.sync_copy(user.at[pl.ds(base, _BPW)], idx_u)
    pltpu.sync_copy(sd_in.at[pl.ds(base, _BPW)], sd_v)
    pltpu.sync_copy(disc_in.at[pl.ds(base, _BPW)], disc_v)
    pltpu.sync_copy(wvec, wv)

    for k in range(_BPW // _L):
        sl = pl.ds(k * _L, _L)
        u = idx_u[sl]
        idx_u[sl] = jnp.minimum(jnp.maximum(u, 0), _USER_NUM - 1)

    wlo = wv[pl.ds(0, _L)]
    whi = wv[pl.ds(_L, _L)]
    wa00, wa01, wa10, wa11 = wlo[0], wlo[1], wlo[2], wlo[3]
    wa20, wa21, wa30, wa31 = wlo[4], wlo[5], wlo[6], wlo[7]
    ba0, ba1, ba2, ba3 = wlo[8], wlo[9], wlo[10], wlo[11]
    wg0, wg1, wg2, wg3 = wlo[12], wlo[13], wlo[14], wlo[15]
    ws0, ws1, ws2, ws3 = whi[0], whi[1], whi[2], whi[3]
    bg0, bs0 = whi[4], whi[5]

    iota = lax.iota(jnp.int32, _L)
    zero = jnp.zeros((_L,), jnp.float32)
    zero_i = jnp.zeros((_L,), jnp.int32)

    def fetch(col, pbuf, sbuf):
        cola = pl.multiple_of(col, _COL)
        pltpu.async_copy(pro_t.at[:, pl.ds(cola, _COL)], pbuf, sem_p)
        pltpu.async_copy(stu_t.at[:, pl.ds(cola, _COL)], sbuf, sem_s)

    def drain(pbuf, sbuf):
        pltpu.make_async_copy(
            pro_t.at[:, pl.ds(0, _COL)], pbuf, sem_p).wait()
        pltpu.make_async_copy(
            stu_t.at[:, pl.ds(0, _COL)], sbuf, sem_s).wait()

    pbufs = (pb0, pb1, pb2, pb3, pb4, pb5, pb6, pb7, pb8, pb9, pb10, pb11, pb12, pb13)
    sbufs = (sb0, sb1, sb2, sb3, sb4, sb5, sb6, sb7, sb8, sb9, sb10, sb11, sb12, sb13)
    _R = 14

    u_vec0 = idx_u[pl.ds(0, _L)]
    col_vec0 = (u_vec0 // _COL) * _COL
    for j in range(_R - 1):
        fetch(col_vec0[j], pbufs[j], sbufs[j])

    def group(g, u_vec):
        sl = pl.ds(g * _L, _L)
        col_vec = (u_vec // _COL) * _COL
        off_vec = u_vec - col_vec
        gn = jnp.minimum(g + 1, _NG - 1)
        u_next = idx_u[pl.ds(gn * _L, _L)]
        col_next = (u_next // _COL) * _COL
        for j in range(_L):
            if j + _R - 1 < _L:
                fetch(col_vec[j + _R - 1], pbufs[(j + _R - 1) % _R],
                      sbufs[(j + _R - 1) % _R])
            else:
                fetch(col_next[j + _R - 1 - _L], pbufs[(j + _R - 1) % _R],
                      sbufs[(j + _R - 1) % _R])
            pbuf, sbuf = pbufs[j % _R], sbufs[j % _R]
            drain(pbuf, sbuf)
            offj = zero_i + off_vec[j]
            stp[j, pl.ds(0, _L)] = plsc.load_gather(pbuf, [iota, offj])
            stp[j, pl.ds(_L, _L)] = plsc.load_gather(pbuf, [iota + _L, offj])
            sts[j, pl.ds(0, _L)] = plsc.load_gather(sbuf, [iota, offj])
            sts[j, pl.ds(_L, _L)] = plsc.load_gather(sbuf, [iota + _L, offj])

        p0 = zero
        p1 = zero
        s0 = zero
        s1 = zero
        for d in range(0, _D, 2):
            cd0 = jnp.full((_L,), d, jnp.int32)
            cd1 = jnp.full((_L,), d + 1, jnp.int32)
            p0 = p0 + _sigmoid(plsc.load_gather(stp, [iota, cd0]))
            p1 = p1 + _sigmoid(plsc.load_gather(stp, [iota, cd1]))
            s0 = s0 + _sigmoid(plsc.load_gather(sts, [iota, cd0]))
            s1 = s1 + _sigmoid(plsc.load_gather(sts, [iota, cd1]))
        spro = p0 + p1
        sstu = s0 + s1

        sd = sd_v[sl]
        disc = disc_v[sl]
        o = _sigmoid(disc * (spro - sd))
        m_stu = sstu * (1.0 / _D)
        m_diff = sd * (1.0 / _D)
        a0 = _sigmoid(wa00 * m_stu + wa01 * m_diff + ba0)
        a1 = _sigmoid(wa10 * m_stu + wa11 * m_diff + ba1)
        a2 = _sigmoid(wa20 * m_stu + wa21 * m_diff + ba2)
        a3 = _sigmoid(wa30 * m_stu + wa31 * m_diff + ba3)
        gv = _sigmoid(wg0 * a0 + wg1 * a1 + wg2 * a2 + wg3 * a3 + bg0)
        sv = _sigmoid(ws0 * a0 + ws1 * a1 + ws2 * a2 + ws3 * a3 + bs0)
        o_v[sl] = (1.0 - sv) * o + gv * (1.0 - o)
        a0_v[sl] = a0
        a1_v[sl] = a1
        a2_v[sl] = a2
        a3_v[sl] = a3
        return u_next

    lax.fori_loop(0, _NG, group, u_vec0)

    for j in range(_R - 1):
        drain(pbufs[j], sbufs[j])

    pltpu.sync_copy(o_v, out_o.at[pl.ds(base, _BPW)])
    pltpu.sync_copy(a0_v, out_a0.at[pl.ds(base, _BPW)])
    pltpu.sync_copy(a1_v, out_a1.at[pl.ds(base, _BPW)])
    pltpu.sync_copy(a2_v, out_a2.at[pl.ds(base, _BPW)])
    pltpu.sync_copy(a3_v, out_a3.at[pl.ds(base, _BPW)])


@functools.partial(
    pl.kernel,
    mesh=plsc.VectorSubcoreMesh(core_axis_name="c", subcore_axis_name="s"),
    compiler_params=pltpu.CompilerParams(
        needs_layout_passes=False, disable_bounds_checks=True),
    out_type=[jax.ShapeDtypeStruct((_B,), jnp.float32)] * 5,
    scratch_types=[
        pltpu.VMEM((_BPW,), jnp.int32),
        *[pltpu.VMEM((_D, _COL), jnp.float32)] * 28,
        pltpu.VMEM((_L, _D), jnp.float32),
        pltpu.VMEM((_L, _D), jnp.float32),
        pltpu.VMEM((_BPW,), jnp.float32),
        pltpu.VMEM((_BPW,), jnp.float32),
        pltpu.VMEM((_BPW,), jnp.float32),
        pltpu.VMEM((_BPW,), jnp.float32),
        pltpu.VMEM((_BPW,), jnp.float32),
        pltpu.VMEM((_BPW,), jnp.float32),
        pltpu.VMEM((_BPW,), jnp.float32),
        pltpu.VMEM((2 * _L,), jnp.float32),
        pltpu.SemaphoreType.DMA,
        pltpu.SemaphoreType.DMA,
    ],
)
def _user_sc(*refs):
    _user_body(*refs)


def kernel(user, item, pro_w, stu_w, diff_w, exerk_w,
           W_aff, b_aff, W_g, b_g, W_s, b_s):
    user = user.astype(jnp.int32)
    item = item.astype(jnp.int32)
    wvec = jnp.concatenate([
        W_aff.reshape(-1), b_aff, W_g.reshape(-1), W_s.reshape(-1),
        b_g, b_s, jnp.zeros((10,), jnp.float32)])
    sd, disc = _item_sc(item, diff_w, exerk_w.reshape(-1))
    o, a0, a1, a2, a3 = _user_sc(user, pro_w.T, stu_w.T, sd, disc, wvec)
    affect = jnp.stack([a0, a1, a2, a3], axis=1)
    return (o, affect)


# speedup vs baseline: 1.0074x; 1.0074x over previous
import functools

import jax
import jax.numpy as jnp
from jax import lax
from jax.experimental import pallas as pl
from jax.experimental.pallas import tpu as pltpu
from jax.experimental.pallas import tpu_sc as plsc

_USER_NUM = 1000000
_ITEM_NUM = 100000
_D = 32
_B = 16384

_NC = 2
_NS = 16
_NW = _NC * _NS
_BPW = _B // _NW
_L = 16
_NG = _BPW // _L
_CHUNK = 128
_NCHUNK = _BPW // _CHUNK
_COL = 128
_LASTCOL = (_USER_NUM // _COL) * _COL


def _sigmoid(x):
    return 1.0 / (1.0 + jnp.exp(-x))


def _item_body(item, diff_w, exerk_w, out_sd, out_disc,
               idx_c, diff_v, ek_v, sd_v, disc_v, sem, sem2):
    wid = lax.axis_index("s") * _NC + lax.axis_index("c")
    base = wid * _BPW

    for c in range(_NCHUNK):
        pltpu.sync_copy(item.at[pl.ds(base + c * _CHUNK, _CHUNK)], idx_c.at[c])
    for c in range(_NCHUNK):
        for k in range(_CHUNK // _L):
            sl = pl.ds(k * _L, _L)
            v = idx_c[c, sl]
            idx_c[c, sl] = jnp.minimum(jnp.maximum(v, 0), _ITEM_NUM - 1)

    copies = []
    for c in range(_NCHUNK):
        rs = pl.ds(c * _CHUNK, _CHUNK)
        copies.append(pltpu.async_copy(diff_w.at[idx_c.at[c]],
                                       diff_v.at[rs], sem))
        copies.append(pltpu.async_copy(exerk_w.at[idx_c.at[c]],
                                       ek_v.at[rs], sem2))
    for cp in copies:
        cp.wait()

    iota = lax.iota(jnp.int32, _L)
    zero = jnp.zeros((_L,), jnp.float32)

    def group(g, carry):
        sl = pl.ds(g * _L, _L)
        rid = g * _L + iota
        s0 = zero
        s1 = zero
        s2 = zero
        s3 = zero
        for d in range(0, _D, 4):
            s0 = s0 + _sigmoid(plsc.load_gather(
                diff_v, [rid, jnp.full((_L,), d, jnp.int32)]))
            s1 = s1 + _sigmoid(plsc.load_gather(
                diff_v, [rid, jnp.full((_L,), d + 1, jnp.int32)]))
            s2 = s2 + _sigmoid(plsc.load_gather(
                diff_v, [rid, jnp.full((_L,), d + 2, jnp.int32)]))
            s3 = s3 + _sigmoid(plsc.load_gather(
                diff_v, [rid, jnp.full((_L,), d + 3, jnp.int32)]))
        sd_v[sl] = (s0 + s1) + (s2 + s3)
        disc_v[sl] = 2.0 * _sigmoid(ek_v[sl])
        return carry

    lax.fori_loop(0, _NG, group, 0)

    pltpu.sync_copy(sd_v, out_sd.at[pl.ds(base, _BPW)])
    pltpu.sync_copy(disc_v, out_disc.at[pl.ds(base, _BPW)])


@functools.partial(
    pl.kernel,
    mesh=plsc.VectorSubcoreMesh(core_axis_name="c", subcore_axis_name="s"),
    compiler_params=pltpu.CompilerParams(
        needs_layout_passes=False, use_tc_tiling_on_sc=False),
    out_type=[
        jax.ShapeDtypeStruct((_B,), jnp.float32),
        jax.ShapeDtypeStruct((_B,), jnp.float32),
    ],
    scratch_types=[
        pltpu.VMEM((_NCHUNK, _CHUNK), jnp.int32),
        pltpu.VMEM((_BPW, _D), jnp.float32),
        pltpu.VMEM((_BPW,), jnp.float32),
        pltpu.VMEM((_BPW,), jnp.float32),
        pltpu.VMEM((_BPW,), jnp.float32),
        pltpu.SemaphoreType.DMA,
        pltpu.SemaphoreType.DMA,
    ],
)
def _item_sc(*refs):
    _item_body(*refs)


def _user_body(user, pro_t, stu_t, sd_in, disc_in, wvec,
               out_o, out_a0, out_a1, out_a2, out_a3,
               idx_u, pb0, pb1, pb2, pb3, pb4, pb5, pb6, pb7,
               pb8, pb9, pb10, pb11,
               sb0, sb1, sb2, sb3, sb4, sb5, sb6, sb7,
               sb8, sb9, sb10, sb11, stp, sts,
               sd_v, disc_v, o_v, a0_v, a1_v, a2_v, a3_v, wv,
               sem_p, sem_s):
    wid = lax.axis_index("s") * _NC + lax.axis_index("c")
    base = wid * _BPW

    pltpu.sync_copy(user.at[pl.ds(base, _BPW)], idx_u)
    pltpu.sync_copy(sd_in.at[pl.ds(base, _BPW)], sd_v)
    pltpu.sync_copy(disc_in.at[pl.ds(base, _BPW)], disc_v)
    pltpu.sync_copy(wvec, wv)

    for k in range(_BPW // _L):
        sl = pl.ds(k * _L, _L)
        u = idx_u[sl]
        idx_u[sl] = jnp.minimum(jnp.maximum(u, 0), _USER_NUM - 1)

    wlo = wv[pl.ds(0, _L)]
    whi = wv[pl.ds(_L, _L)]
    wa00, wa01, wa10, wa11 = wlo[0], wlo[1], wlo[2], wlo[3]
    wa20, wa21, wa30, wa31 = wlo[4], wlo[5], wlo[6], wlo[7]
    ba0, ba1, ba2, ba3 = wlo[8], wlo[9], wlo[10], wlo[11]
    wg0, wg1, wg2, wg3 = wlo[12], wlo[13], wlo[14], wlo[15]
    ws0, ws1, ws2, ws3 = whi[0], whi[1], whi[2], whi[3]
    bg0, bs0 = whi[4], whi[5]

    iota = lax.iota(jnp.int32, _L)
    zero = jnp.zeros((_L,), jnp.float32)
    zero_i = jnp.zeros((_L,), jnp.int32)

    def fetch(col, pbuf, sbuf):
        cola = pl.multiple_of(col, _COL)
        pltpu.async_copy(pro_t.at[:, pl.ds(cola, _COL)], pbuf, sem_p)
        pltpu.async_copy(stu_t.at[:, pl.ds(cola, _COL)], sbuf, sem_s)

    def drain(pbuf, sbuf):
        pltpu.make_async_copy(
            pro_t.at[:, pl.ds(0, _COL)], pbuf, sem_p).wait()
        pltpu.make_async_copy(
            stu_t.at[:, pl.ds(0, _COL)], sbuf, sem_s).wait()

    pbufs = (pb0, pb1, pb2, pb3, pb4, pb5, pb6, pb7, pb8, pb9, pb10, pb11)
    sbufs = (sb0, sb1, sb2, sb3, sb4, sb5, sb6, sb7, sb8, sb9, sb10, sb11)
    _R = 12

    u_vec0 = idx_u[pl.ds(0, _L)]
    col_vec0 = (u_vec0 // _COL) * _COL
    for j in range(_R - 1):
        fetch(col_vec0[j], pbufs[j], sbufs[j])

    def group(g, u_vec):
        sl = pl.ds(g * _L, _L)
        col_vec = (u_vec // _COL) * _COL
        off_vec = u_vec - col_vec
        gn = jnp.minimum(g + 1, _NG - 1)
        u_next = idx_u[pl.ds(gn * _L, _L)]
        col_next = (u_next // _COL) * _COL
        for j in range(_L):
            if j + _R - 1 < _L:
                fetch(col_vec[j + _R - 1], pbufs[(j + _R - 1) % _R],
                      sbufs[(j + _R - 1) % _R])
            else:
                fetch(col_next[j + _R - 1 - _L], pbufs[(j + _R - 1) % _R],
                      sbufs[(j + _R - 1) % _R])
            pbuf, sbuf = pbufs[j % _R], sbufs[j % _R]
            drain(pbuf, sbuf)
            offj = zero_i + off_vec[j]
            stp[j, pl.ds(0, _L)] = plsc.load_gather(pbuf, [iota, offj])
            stp[j, pl.ds(_L, _L)] = plsc.load_gather(pbuf, [iota + _L, offj])
            sts[j, pl.ds(0, _L)] = plsc.load_gather(sbuf, [iota, offj])
            sts[j, pl.ds(_L, _L)] = plsc.load_gather(sbuf, [iota + _L, offj])

        p0 = zero
        p1 = zero
        s0 = zero
        s1 = zero
        for d in range(0, _D, 2):
            cd0 = jnp.full((_L,), d, jnp.int32)
            cd1 = jnp.full((_L,), d + 1, jnp.int32)
            p0 = p0 + _sigmoid(plsc.load_gather(stp, [iota, cd0]))
            p1 = p1 + _sigmoid(plsc.load_gather(stp, [iota, cd1]))
            s0 = s0 + _sigmoid(plsc.load_gather(sts, [iota, cd0]))
            s1 = s1 + _sigmoid(plsc.load_gather(sts, [iota, cd1]))
        spro = p0 + p1
        sstu = s0 + s1

        sd = sd_v[sl]
        disc = disc_v[sl]
        o = _sigmoid(disc * (spro - sd))
        m_stu = sstu * (1.0 / _D)
        m_diff = sd * (1.0 / _D)
        a0 = _sigmoid(wa00 * m_stu + wa01 * m_diff + ba0)
        a1 = _sigmoid(wa10 * m_stu + wa11 * m_diff + ba1)
        a2 = _sigmoid(wa20 * m_stu + wa21 * m_diff + ba2)
        a3 = _sigmoid(wa30 * m_stu + wa31 * m_diff + ba3)
        gv = _sigmoid(wg0 * a0 + wg1 * a1 + wg2 * a2 + wg3 * a3 + bg0)
        sv = _sigmoid(ws0 * a0 + ws1 * a1 + ws2 * a2 + ws3 * a3 + bs0)
        o_v[sl] = (1.0 - sv) * o + gv * (1.0 - o)
        a0_v[sl] = a0
        a1_v[sl] = a1
        a2_v[sl] = a2
        a3_v[sl] = a3
        return u_next

    lax.fori_loop(0, _NG, group, u_vec0)

    for j in range(_R - 1):
        drain(pbufs[j], sbufs[j])

    pltpu.sync_copy(o_v, out_o.at[pl.ds(base, _BPW)])
    pltpu.sync_copy(a0_v, out_a0.at[pl.ds(base, _BPW)])
    pltpu.sync_copy(a1_v, out_a1.at[pl.ds(base, _BPW)])
    pltpu.sync_copy(a2_v, out_a2.at[pl.ds(base, _BPW)])
    pltpu.sync_copy(a3_v, out_a3.at[pl.ds(base, _BPW)])


@functools.partial(
    pl.kernel,
    mesh=plsc.VectorSubcoreMesh(core_axis_name="c", subcore_axis_name="s"),
    compiler_params=pltpu.CompilerParams(
        needs_layout_passes=False, disable_bounds_checks=True),
    out_type=[jax.ShapeDtypeStruct((_B,), jnp.float32)] * 5,
    scratch_types=[
        pltpu.VMEM((_BPW,), jnp.int32),
        *[pltpu.VMEM((_D, _COL), jnp.float32)] * 24,
        pltpu.VMEM((_L, _D), jnp.float32),
        pltpu.VMEM((_L, _D), jnp.float32),
        pltpu.VMEM((_BPW,), jnp.float32),
        pltpu.VMEM((_BPW,), jnp.float32),
        pltpu.VMEM((_BPW,), jnp.float32),
        pltpu.VMEM((_BPW,), jnp.float32),
        pltpu.VMEM((_BPW,), jnp.float32),
        pltpu.VMEM((_BPW,), jnp.float32),
        pltpu.VMEM((_BPW,), jnp.float32),
        pltpu.VMEM((2 * _L,), jnp.float32),
        pltpu.SemaphoreType.DMA,
        pltpu.SemaphoreType.DMA,
    ],
)
def _user_sc(*refs):
    _user_body(*refs)


def kernel(user, item, pro_w, stu_w, diff_w, exerk_w,
           W_aff, b_aff, W_g, b_g, W_s, b_s):
    user = user.astype(jnp.int32)
    item = item.astype(jnp.int32)
    wvec = jnp.concatenate([
        W_aff.reshape(-1), b_aff, W_g.reshape(-1), W_s.reshape(-1),
        b_g, b_s, jnp.zeros((10,), jnp.float32)])
    sd, disc = _item_sc(item, diff_w, exerk_w.reshape(-1))
    o, a0, a1, a2, a3 = _user_sc(user, pro_w.T, stu_w.T, sd, disc, wvec)
    affect = jnp.stack([a0, a1, a2, a3], axis=1)
    return (o, affect)
